# Initial kernel scaffold; baseline (speedup 1.0000x reference)
import functools

import jax
import jax.numpy as jnp
from jax import lax
from jax.experimental import pallas as pl
from jax.experimental.pallas import tpu as pltpu
from jax.experimental.pallas import tpu_sc as plsc

N = 10000
E = 320000
IN = 128
HID = 64
OUT = 32

NC = 2
NS = 16
L = 16
NW = NC * NS
EPW = E // NW
CH = 80
NCHUNK = EPW // CH
ROWS_PER_TILE = N // NS
ZCH = 125


def _proj1_body(x_ref, w_ref, a2_ref, haug_ref, als_ref):
    h = jnp.dot(x_ref[...], w_ref[...], preferred_element_type=jnp.float32)
    r = h.shape[0]
    ones = jnp.ones((r, 1), jnp.float32)
    pad = jnp.zeros((r, 80 - HID - 1), jnp.float32)
    haug_ref[...] = jnp.concatenate([h, ones, pad], axis=1)
    als_ref[...] = lax.dot_general(a2_ref[...], h, (((1,), (1,)), ((), ())),
                                   preferred_element_type=jnp.float32)


def _combine1_body(agg_ref, b1_ref, w2_ref, a2_ref, h2aug_ref, als2_ref):
    p = agg_ref[...]
    s = p[0] + p[1]
    den = s[:, HID:HID + 1]
    h1 = s[:, :HID] / (den + 1e-16) + b1_ref[...]
    h2 = jnp.dot(h1, w2_ref[...], preferred_element_type=jnp.float32)
    r = h2.shape[0]
    ones = jnp.ones((r, 1), jnp.float32)
    pad = jnp.zeros((r, 48 - OUT - 1), jnp.float32)
    h2aug_ref[...] = jnp.concatenate([h2, ones, pad], axis=1)
    als2_ref[...] = lax.dot_general(a2_ref[...], h2, (((1,), (1,)), ((), ())),
                                    preferred_element_type=jnp.float32)


def _head_body(agg_ref, b2_ref, wlin_ref, blin_ref, z_ref, xr_ref):
    p = agg_ref[...]
    s = p[0] + p[1]
    den = s[:, OUT:OUT + 1]
    h = s[:, :OUT] / (den + 1e-16) + b2_ref[...]
    nrm = jnp.sqrt(jnp.sum(h * h, axis=1, keepdims=True))
    z = h / jnp.maximum(nrm, 1e-12)
    z_ref[...] = z
    zw = jnp.dot(z, wlin_ref[...], preferred_element_type=jnp.float32)
    zw = zw + blin_ref[...]
    xr_ref[...] = jnp.where(zw > 0, zw, jnp.exp(zw) - 1.0)


def _adj_body(zi_ref, zj_ref, out_ref):
    g = lax.dot_general(zi_ref[...], zj_ref[...], (((1,), (1,)), ((), ())),
                        preferred_element_type=jnp.float32)
    out_ref[...] = 1.0 / (1.0 + jnp.exp(-g))


def _edge_body(w, haug_hbm, als_hbm, edge_hbm, out_hbm,
               src2d, dst2d, alsrc_v, aldst_v, exbuf, rows_v, zbuf, sem):
    cid = lax.axis_index("c")
    sid = lax.axis_index("s")
    wid = cid * NS + sid
    base = wid * EPW

    def zero_zbuf(i, _):
        for k in range(w // L):
            zbuf[i, pl.ds(k * L, L)] = jnp.zeros((L,), jnp.float32)
        return 0
    lax.fori_loop(0, ZCH, zero_zbuf, 0)

    def run(agg_sp):
        def zero_agg(i, _):
            pltpu.sync_copy(zbuf, agg_sp.at[pl.ds(sid * ROWS_PER_TILE + i * ZCH,
                                                  ZCH)])
            return 0
        lax.fori_loop(0, ROWS_PER_TILE // ZCH, zero_agg, 0)

        def load_idx(i, _):
            pltpu.sync_copy(edge_hbm.at[0, pl.ds(base + i * CH, CH)],
                            src2d.at[i])
            pltpu.sync_copy(edge_hbm.at[1, pl.ds(base + i * CH, CH)],
                            dst2d.at[i])
            return 0
        lax.fori_loop(0, NCHUNK, load_idx, 0)

        pltpu.sync_copy(als_hbm.at[0], alsrc_v)
        pltpu.sync_copy(als_hbm.at[1], aldst_v)

        def maxv(i, carry):
            ms, md = carry
            ms = jnp.maximum(ms, alsrc_v[pl.ds(i * L, L)])
            md = jnp.maximum(md, aldst_v[pl.ds(i * L, L)])
            return ms, md
        neg = jnp.full((L,), -3e38, jnp.float32)
        ms, md = lax.fori_loop(0, N // L, maxv, (neg, neg))
        c = jnp.max(ms) + jnp.max(md)
        bound = jnp.where(c > 0, c, 0.2 * c)

        plsc.subcore_barrier()

        def chunk(j, _):
            for k in range(CH // L):
                si = src2d[j, pl.ds(k * L, L)]
                di = dst2d[j, pl.ds(k * L, L)]
                sv = plsc.load_gather(alsrc_v, [si])
                dv = plsc.load_gather(aldst_v, [di])
                e = sv + dv
                e = jnp.where(e > 0, e, 0.2 * e)
                exbuf[pl.ds(k * L, L)] = jnp.exp(e - bound)

            pltpu.async_copy(haug_hbm.at[src2d.at[j]], rows_v, sem).wait()

            def scale(e_i, _):
                s = exbuf[e_i]
                for k in range(w // L):
                    rows_v[e_i, pl.ds(k * L, L)] = (
                        rows_v[e_i, pl.ds(k * L, L)] * s)
                return 0
            lax.fori_loop(0, CH, scale, 0)

            pltpu.sync_copy(rows_v, agg_sp.at[dst2d.at[j]], add=True)
            return 0

        lax.fori_loop(0, NCHUNK, chunk, 0)

        plsc.subcore_barrier()

        pltpu.sync_copy(agg_sp.at[pl.ds(sid * ROWS_PER_TILE, ROWS_PER_TILE)],
                        out_hbm.at[cid, pl.ds(sid * ROWS_PER_TILE,
                                              ROWS_PER_TILE)])

    pl.run_scoped(run, pltpu.VMEM_SHARED((N, w), jnp.float32))


def _edge_phase(w, haug, als, edge_index):
    mesh = plsc.VectorSubcoreMesh(core_axis_name="c", subcore_axis_name="s",
                                  num_cores=NC, num_subcores=NS)
    fn = pl.kernel(
        functools.partial(_edge_body, w),
        out_type=jax.ShapeDtypeStruct((NC, N, w), jnp.float32),
        mesh=mesh,
        scratch_types=[
            pltpu.VMEM((NCHUNK, CH), jnp.int32),
            pltpu.VMEM((NCHUNK, CH), jnp.int32),
            pltpu.VMEM((N,), jnp.float32),
            pltpu.VMEM((N,), jnp.float32),
            pltpu.VMEM((CH,), jnp.float32),
            pltpu.VMEM((CH, w), jnp.float32),
            pltpu.VMEM((ZCH, w), jnp.float32),
            pltpu.SemaphoreType.DMA,
        ],
        name=f"gat_edge_w{w}",
    )
    return fn(haug, als, edge_index)


def kernel(x, edge_index, W1, a_src1, a_dst1, b1, W2, a_src2, a_dst2, b2,
           Wlin, blin):
    R = 400
    G = N // R

    a2_1 = jnp.concatenate([a_src1.reshape(1, HID), a_dst1.reshape(1, HID)], 0)
    a2_2 = jnp.concatenate([a_src2.reshape(1, OUT), a_dst2.reshape(1, OUT)], 0)

    haug, als1 = pl.pallas_call(
        _proj1_body,
        grid=(G,),
        in_specs=[
            pl.BlockSpec((R, IN), lambda i: (i, 0)),
            pl.BlockSpec((IN, HID), lambda i: (0, 0)),
            pl.BlockSpec((2, HID), lambda i: (0, 0)),
        ],
        out_specs=[
            pl.BlockSpec((R, 80), lambda i: (i, 0)),
            pl.BlockSpec((2, R), lambda i: (0, i)),
        ],
        out_shape=[
            jax.ShapeDtypeStruct((N, 80), jnp.float32),
            jax.ShapeDtypeStruct((2, N), jnp.float32),
        ],
    )(x, W1, a2_1)

    agg1 = _edge_phase(80, haug, als1, edge_index)

    h2aug, als2 = pl.pallas_call(
        _combine1_body,
        grid=(G,),
        in_specs=[
            pl.BlockSpec((NC, R, 80), lambda i: (0, i, 0)),
            pl.BlockSpec((1, HID), lambda i: (0, 0)),
            pl.BlockSpec((HID, OUT), lambda i: (0, 0)),
            pl.BlockSpec((2, OUT), lambda i: (0, 0)),
        ],
        out_specs=[
            pl.BlockSpec((R, 48), lambda i: (i, 0)),
            pl.BlockSpec((2, R), lambda i: (0, i)),
        ],
        out_shape=[
            jax.ShapeDtypeStruct((N, 48), jnp.float32),
            jax.ShapeDtypeStruct((2, N), jnp.float32),
        ],
    )(agg1, b1.reshape(1, HID), W2, a2_2)

    agg2 = _edge_phase(48, h2aug, als2, edge_index)

    z, x_ = pl.pallas_call(
        _head_body,
        grid=(G,),
        in_specs=[
            pl.BlockSpec((NC, R, 48), lambda i: (0, i, 0)),
            pl.BlockSpec((1, OUT), lambda i: (0, 0)),
            pl.BlockSpec((OUT, IN), lambda i: (0, 0)),
            pl.BlockSpec((1, IN), lambda i: (0, 0)),
        ],
        out_specs=[
            pl.BlockSpec((R, OUT), lambda i: (i, 0)),
            pl.BlockSpec((R, IN), lambda i: (i, 0)),
        ],
        out_shape=[
            jax.ShapeDtypeStruct((N, OUT), jnp.float32),
            jax.ShapeDtypeStruct((N, IN), jnp.float32),
        ],
    )(agg2, b2.reshape(1, OUT), Wlin, blin.reshape(1, IN))

    B = 256
    GA = pl.cdiv(N, B)
    zp = jnp.pad(z, ((0, GA * B - N), (0, 0)))
    a_pred = pl.pallas_call(
        _adj_body,
        grid=(GA, GA),
        in_specs=[
            pl.BlockSpec((B, OUT), lambda i, j: (i, 0)),
            pl.BlockSpec((B, OUT), lambda i, j: (j, 0)),
        ],
        out_specs=pl.BlockSpec((B, B), lambda i, j: (i, j)),
        out_shape=jax.ShapeDtypeStruct((N, N), jnp.float32),
    )(zp, zp)

    return (a_pred, z, x_)


# speedup vs baseline: 11.3310x; 11.3310x over previous
import functools

import jax
import jax.numpy as jnp
from jax import lax
from jax.experimental import pallas as pl
from jax.experimental.pallas import tpu as pltpu
from jax.experimental.pallas import tpu_sc as plsc

N = 10000
E = 320000
IN = 128
HID = 64
OUT = 32

NC = 2
NS = 16
L = 16
NW = NC * NS
EPW = E // NW
CH = 80
NCHUNK = EPW // CH
RPT = 624
ZCH = 16


def _proj1_body(x_ref, w_ref, a2_ref, haug_ref, alsrc_ref, aldst_ref,
                bnd_ref):
    h = jnp.dot(x_ref[...], w_ref[...], preferred_element_type=jnp.float32)
    r = h.shape[0]
    ones = jnp.ones((r, 1), jnp.float32)
    pad = jnp.zeros((r, 80 - HID - 1), jnp.float32)
    haug_ref[...] = jnp.concatenate([h, ones, pad], axis=1)
    als = jnp.dot(h, a2_ref[...], preferred_element_type=jnp.float32)
    alsrc_ref[...] = als[:, 0:1]
    aldst_ref[...] = als[:, 1:2]
    _accum_bound(als, bnd_ref)


def _accum_bound(als, bnd_ref):
    @pl.when(pl.program_id(0) == 0)
    def _():
        bnd_ref[...] = jnp.full((1, 16), -3e38, jnp.float32)
    m = jnp.max(als[:, 0])
    d = jnp.max(als[:, 1])
    iot = lax.broadcasted_iota(jnp.int32, (1, 16), 1)
    upd = jnp.where(iot == 0, m, jnp.where(iot == 1, d, -3e38))
    bnd_ref[...] = jnp.maximum(bnd_ref[...], upd)


def _combine1_body(agg_ref, b1_ref, w2_ref, a2_ref, h2aug_ref, alsrc_ref,
                   aldst_ref, bnd_ref):
    p = agg_ref[...]
    s = p[0] + p[1]
    den = s[:, HID:HID + 1]
    h1 = s[:, :HID] / (den + 1e-16) + b1_ref[...]
    h2 = jnp.dot(h1, w2_ref[...], preferred_element_type=jnp.float32)
    r = h2.shape[0]
    ones = jnp.ones((r, 1), jnp.float32)
    pad = jnp.zeros((r, 48 - OUT - 1), jnp.float32)
    h2aug_ref[...] = jnp.concatenate([h2, ones, pad], axis=1)
    als = jnp.dot(h2, a2_ref[...], preferred_element_type=jnp.float32)
    alsrc_ref[...] = als[:, 0:1]
    aldst_ref[...] = als[:, 1:2]
    _accum_bound(als, bnd_ref)


def _head_body(agg_ref, b2_ref, wlin_ref, blin_ref, z_ref, xr_ref):
    p = agg_ref[...]
    s = p[0] + p[1]
    den = s[:, OUT:OUT + 1]
    h = s[:, :OUT] / (den + 1e-16) + b2_ref[...]
    nrm = jnp.sqrt(jnp.sum(h * h, axis=1, keepdims=True))
    z = h / jnp.maximum(nrm, 1e-12)
    z_ref[...] = z
    zw = jnp.dot(z, wlin_ref[...], preferred_element_type=jnp.float32)
    zw = zw + blin_ref[...]
    xr_ref[...] = jnp.where(zw > 0, zw, jnp.exp(zw) - 1.0)


def _adj_body(zi_ref, zj_ref, out_ref):
    g = lax.dot_general(zi_ref[...], zj_ref[...], (((1,), (1,)), ((), ())),
                        preferred_element_type=jnp.float32)
    out_ref[...] = 1.0 / (1.0 + jnp.exp(-g))


def _edge_body(w, haug_hbm, alsrc_hbm, aldst_hbm, bnd_hbm, esrc_hbm, edst_hbm,
               out_hbm, src2d, dst2d, alsrc_v, aldst_v, bndv, exbuf, rows_v,
               zbuf, agg_sp, sem):
    cid = lax.axis_index("c")
    sid = lax.axis_index("s")
    wid = cid * NS + sid
    base = wid * EPW

    def zero_zbuf(i, _):
        for k in range(w // L):
            zbuf[i, pl.ds(k * L, L)] = jnp.zeros((L,), jnp.float32)
        return 0
    lax.fori_loop(0, ZCH, zero_zbuf, 0)

    if True:
        def zero_agg(i, _):
            pltpu.sync_copy(zbuf, agg_sp.at[pl.ds(sid * RPT + i * ZCH, ZCH)])
            return 0
        lax.fori_loop(0, RPT // ZCH, zero_agg, 0)

        @pl.when(sid == NS - 1)
        def _():
            pltpu.sync_copy(zbuf, agg_sp.at[pl.ds(NS * RPT, N - NS * RPT)])

        def load_idx(i, _):
            pltpu.sync_copy(esrc_hbm.at[pl.ds(base + i * CH, CH)],
                            src2d.at[i])
            pltpu.sync_copy(edst_hbm.at[pl.ds(base + i * CH, CH)],
                            dst2d.at[i])
            return 0
        lax.fori_loop(0, NCHUNK, load_idx, 0)

        pltpu.sync_copy(alsrc_hbm, alsrc_v)
        pltpu.sync_copy(aldst_hbm, aldst_v)
        pltpu.sync_copy(bnd_hbm, bndv)
        bv = bndv[...]
        c = bv[0] + bv[1]
        bound = jnp.where(c > 0, c, 0.2 * c)

        plsc.subcore_barrier()

        def chunk(j, _):
            for k in range(CH // L):
                si = src2d[j, pl.ds(k * L, L)]
                di = dst2d[j, pl.ds(k * L, L)]
                sv = plsc.load_gather(alsrc_v, [si])
                dv = plsc.load_gather(aldst_v, [di])
                e = sv + dv
                e = jnp.where(e > 0, e, 0.2 * e)
                exbuf[pl.ds(k * L, L)] = jnp.exp(e - bound)

            pltpu.async_copy(haug_hbm.at[src2d.at[j]], rows_v, sem).wait()

            def scale(g, _):
                exv = exbuf[pl.ds(g * L, L)]
                for l in range(L):
                    s = exv[l]
                    ei = g * L + l
                    for k in range(w // L):
                        rows_v[ei, pl.ds(k * L, L)] = (
                            rows_v[ei, pl.ds(k * L, L)] * s)
                return 0
            lax.fori_loop(0, CH // L, scale, 0)

            pltpu.sync_copy(rows_v, agg_sp.at[dst2d.at[j]], add=True)
            return 0

        lax.fori_loop(0, NCHUNK, chunk, 0)

        plsc.subcore_barrier()

        pltpu.sync_copy(agg_sp.at[pl.ds(sid * RPT, RPT)],
                        out_hbm.at[cid, pl.ds(sid * RPT, RPT)])

        @pl.when(sid == NS - 1)
        def _():
            pltpu.sync_copy(agg_sp.at[pl.ds(NS * RPT, N - NS * RPT)],
                            out_hbm.at[cid, pl.ds(NS * RPT, N - NS * RPT)])


def _edge_phase(w, haug, alsrc, aldst, bnd, esrc, edst):
    mesh = plsc.VectorSubcoreMesh(core_axis_name="c", subcore_axis_name="s",
                                  num_cores=NC, num_subcores=NS)
    fn = pl.kernel(
        functools.partial(_edge_body, w),
        out_type=jax.ShapeDtypeStruct((NC, N, w), jnp.float32),
        mesh=mesh,
        scratch_types=[
            pltpu.VMEM((NCHUNK, CH), jnp.int32),
            pltpu.VMEM((NCHUNK, CH), jnp.int32),
            pltpu.VMEM((N,), jnp.float32),
            pltpu.VMEM((N,), jnp.float32),
            pltpu.VMEM((L,), jnp.float32),
            pltpu.VMEM((CH,), jnp.float32),
            pltpu.VMEM((CH, w), jnp.float32),
            pltpu.VMEM((ZCH, w), jnp.float32),
            pltpu.VMEM_SHARED((N, w), jnp.float32),
            pltpu.SemaphoreType.DMA,
        ],
        compiler_params=pltpu.CompilerParams(needs_layout_passes=False,
                                             use_tc_tiling_on_sc=False),
        name=f"gat_edge_w{w}",
    )
    return fn(haug, alsrc, aldst, bnd, esrc, edst)


def kernel(x, edge_index, W1, a_src1, a_dst1, b1, W2, a_src2, a_dst2, b2,
           Wlin, blin):
    R = 400
    G = N // R

    a2_1 = jnp.stack([a_src1.reshape(HID), a_dst1.reshape(HID)], axis=1)
    a2_2 = jnp.stack([a_src2.reshape(OUT), a_dst2.reshape(OUT)], axis=1)

    haug, alsrc1, aldst1, bnd1 = pl.pallas_call(
        _proj1_body,
        grid=(G,),
        in_specs=[
            pl.BlockSpec((R, IN), lambda i: (i, 0)),
            pl.BlockSpec((IN, HID), lambda i: (0, 0)),
            pl.BlockSpec((HID, 2), lambda i: (0, 0)),
        ],
        out_specs=[
            pl.BlockSpec((R, 80), lambda i: (i, 0)),
            pl.BlockSpec((R, 1), lambda i: (i, 0)),
            pl.BlockSpec((R, 1), lambda i: (i, 0)),
            pl.BlockSpec((1, 16), lambda i: (0, 0)),
        ],
        out_shape=[
            jax.ShapeDtypeStruct((N, 80), jnp.float32),
            jax.ShapeDtypeStruct((N, 1), jnp.float32),
            jax.ShapeDtypeStruct((N, 1), jnp.float32),
            jax.ShapeDtypeStruct((1, 16), jnp.float32),
        ],
    )(x, W1, a2_1)

    esrc = edge_index[0]
    edst = edge_index[1]
    agg1 = _edge_phase(80, haug, alsrc1.reshape(N), aldst1.reshape(N),
                       bnd1.reshape(16), esrc, edst)

    h2aug, alsrc2, aldst2, bnd2 = pl.pallas_call(
        _combine1_body,
        grid=(G,),
        in_specs=[
            pl.BlockSpec((NC, R, 80), lambda i: (0, i, 0)),
            pl.BlockSpec((1, HID), lambda i: (0, 0)),
            pl.BlockSpec((HID, OUT), lambda i: (0, 0)),
            pl.BlockSpec((OUT, 2), lambda i: (0, 0)),
        ],
        out_specs=[
            pl.BlockSpec((R, 48), lambda i: (i, 0)),
            pl.BlockSpec((R, 1), lambda i: (i, 0)),
            pl.BlockSpec((R, 1), lambda i: (i, 0)),
            pl.BlockSpec((1, 16), lambda i: (0, 0)),
        ],
        out_shape=[
            jax.ShapeDtypeStruct((N, 48), jnp.float32),
            jax.ShapeDtypeStruct((N, 1), jnp.float32),
            jax.ShapeDtypeStruct((N, 1), jnp.float32),
            jax.ShapeDtypeStruct((1, 16), jnp.float32),
        ],
    )(agg1, b1.reshape(1, HID), W2, a2_2)

    agg2 = _edge_phase(48, h2aug, alsrc2.reshape(N), aldst2.reshape(N),
                       bnd2.reshape(16), esrc, edst)

    z, x_ = pl.pallas_call(
        _head_body,
        grid=(G,),
        in_specs=[
            pl.BlockSpec((NC, R, 48), lambda i: (0, i, 0)),
            pl.BlockSpec((1, OUT), lambda i: (0, 0)),
            pl.BlockSpec((OUT, IN), lambda i: (0, 0)),
            pl.BlockSpec((1, IN), lambda i: (0, 0)),
        ],
        out_specs=[
            pl.BlockSpec((R, OUT), lambda i: (i, 0)),
            pl.BlockSpec((R, IN), lambda i: (i, 0)),
        ],
        out_shape=[
            jax.ShapeDtypeStruct((N, OUT), jnp.float32),
            jax.ShapeDtypeStruct((N, IN), jnp.float32),
        ],
    )(agg2, b2.reshape(1, OUT), Wlin, blin.reshape(1, IN))

    B = 256
    GA = pl.cdiv(N, B)
    zp = jnp.pad(z, ((0, GA * B - N), (0, 0)))
    a_pred = pl.pallas_call(
        _adj_body,
        grid=(GA, GA),
        in_specs=[
            pl.BlockSpec((B, OUT), lambda i, j: (i, 0)),
            pl.BlockSpec((B, OUT), lambda i, j: (j, 0)),
        ],
        out_specs=pl.BlockSpec((B, B), lambda i, j: (i, j)),
        out_shape=jax.ShapeDtypeStruct((N, N), jnp.float32),
    )(zp, zp)

    return (a_pred, z, x_)


# speedup vs baseline: 18.1970x; 1.6060x over previous
import functools

import jax
import jax.numpy as jnp
from jax import lax
from jax.experimental import pallas as pl
from jax.experimental.pallas import tpu as pltpu
from jax.experimental.pallas import tpu_sc as plsc

N = 10000
E = 320000
IN = 128
HID = 64
OUT = 32

NC = 2
NS = 16
L = 16
NW = NC * NS
EPW = E // NW
CH = 80
NCHUNK = EPW // CH
RPT = 624
ZCH = 16


def _proj1_body(x_ref, w_ref, a2_ref, haug_ref, alsrc_ref, aldst_ref,
                bnd_ref):
    h = jnp.dot(x_ref[...], w_ref[...], preferred_element_type=jnp.float32)
    r = h.shape[0]
    ones = jnp.ones((r, 1), jnp.float32)
    pad = jnp.zeros((r, 80 - HID - 1), jnp.float32)
    haug_ref[...] = jnp.concatenate([h, ones, pad], axis=1)
    als = jnp.dot(h, a2_ref[...], preferred_element_type=jnp.float32)
    alsrc_ref[...] = als[:, 0:1]
    aldst_ref[...] = als[:, 1:2]
    _accum_bound(als, bnd_ref)


def _accum_bound(als, bnd_ref):
    @pl.when(pl.program_id(0) == 0)
    def _():
        bnd_ref[...] = jnp.full((1, 16), -3e38, jnp.float32)
    m = jnp.max(als[:, 0])
    d = jnp.max(als[:, 1])
    iot = lax.broadcasted_iota(jnp.int32, (1, 16), 1)
    upd = jnp.where(iot == 0, m, jnp.where(iot == 1, d, -3e38))
    bnd_ref[...] = jnp.maximum(bnd_ref[...], upd)


def _combine1_body(agg_ref, b1_ref, w2_ref, a2_ref, h2aug_ref, alsrc_ref,
                   aldst_ref, bnd_ref):
    p = agg_ref[...]
    s = p[0] + p[1]
    den = s[:, HID:HID + 1]
    h1 = s[:, :HID] / (den + 1e-16) + b1_ref[...]
    h2 = jnp.dot(h1, w2_ref[...], preferred_element_type=jnp.float32)
    r = h2.shape[0]
    ones = jnp.ones((r, 1), jnp.float32)
    pad = jnp.zeros((r, 48 - OUT - 1), jnp.float32)
    h2aug_ref[...] = jnp.concatenate([h2, ones, pad], axis=1)
    als = jnp.dot(h2, a2_ref[...], preferred_element_type=jnp.float32)
    alsrc_ref[...] = als[:, 0:1]
    aldst_ref[...] = als[:, 1:2]
    _accum_bound(als, bnd_ref)


def _head_body(agg_ref, b2_ref, wlin_ref, blin_ref, z_ref, xr_ref):
    p = agg_ref[...]
    s = p[0] + p[1]
    den = s[:, OUT:OUT + 1]
    h = s[:, :OUT] / (den + 1e-16) + b2_ref[...]
    nrm = jnp.sqrt(jnp.sum(h * h, axis=1, keepdims=True))
    z = h / jnp.maximum(nrm, 1e-12)
    z_ref[...] = z
    zw = jnp.dot(z, wlin_ref[...], preferred_element_type=jnp.float32)
    zw = zw + blin_ref[...]
    xr_ref[...] = jnp.where(zw > 0, zw, jnp.exp(zw) - 1.0)


def _adj_body(zi_ref, zj_ref, out_ref):
    g = lax.dot_general(zi_ref[...], zj_ref[...], (((1,), (1,)), ((), ())),
                        preferred_element_type=jnp.float32)
    out_ref[...] = 1.0 / (1.0 + jnp.exp(-g))


def _edge_body(w, haug_hbm, alsrc_hbm, aldst_hbm, bnd_hbm, esrc_hbm, edst_hbm,
               out_hbm, src2d, dst2d, alsrc_v, aldst_v, bndv, exbuf, rows_v,
               zbuf, agg_sp, sem):
    cid = lax.axis_index("c")
    sid = lax.axis_index("s")
    wid = cid * NS + sid
    base = wid * EPW

    def zero_zbuf(i, _):
        for k in range(w // L):
            zbuf[i, pl.ds(k * L, L)] = jnp.zeros((L,), jnp.float32)
        return 0
    lax.fori_loop(0, ZCH, zero_zbuf, 0)

    if True:
        def zero_agg(i, _):
            pltpu.sync_copy(zbuf, agg_sp.at[pl.ds(sid * RPT + i * ZCH, ZCH)])
            return 0
        lax.fori_loop(0, RPT // ZCH, zero_agg, 0)

        @pl.when(sid == NS - 1)
        def _():
            pltpu.sync_copy(zbuf, agg_sp.at[pl.ds(NS * RPT, N - NS * RPT)])

        def load_idx(i, _):
            pltpu.sync_copy(esrc_hbm.at[pl.ds(base + i * CH, CH)],
                            src2d.at[i])
            pltpu.sync_copy(edst_hbm.at[pl.ds(base + i * CH, CH)],
                            dst2d.at[i])
            return 0
        lax.fori_loop(0, NCHUNK, load_idx, 0)

        pltpu.sync_copy(alsrc_hbm, alsrc_v)
        pltpu.sync_copy(aldst_hbm, aldst_v)
        pltpu.sync_copy(bnd_hbm, bndv)
        bv = bndv[...]
        c = bv[0] + bv[1]
        bound = jnp.where(c > 0, c, 0.2 * c)

        plsc.subcore_barrier()

        def chunk(j, _):
            for k in range(CH // L):
                si = src2d[j, pl.ds(k * L, L)]
                di = dst2d[j, pl.ds(k * L, L)]
                sv = plsc.load_gather(alsrc_v, [si])
                dv = plsc.load_gather(aldst_v, [di])
                e = sv + dv
                e = jnp.where(e > 0, e, 0.2 * e)
                exbuf[pl.ds(k * L, L)] = jnp.exp(e - bound)

            pltpu.async_copy(haug_hbm.at[src2d.at[j]], rows_v, sem).wait()

            def scale(g, _):
                exv = exbuf[pl.ds(g * L, L)]
                for l in range(L):
                    s = exv[l]
                    ei = g * L + l
                    for k in range(w // L):
                        rows_v[ei, pl.ds(k * L, L)] = (
                            rows_v[ei, pl.ds(k * L, L)] * s)
                return 0
            lax.fori_loop(0, CH // L, scale, 0)

            pltpu.sync_copy(rows_v, agg_sp.at[dst2d.at[j]], add=True)
            return 0

        lax.fori_loop(0, NCHUNK, chunk, 0)

        plsc.subcore_barrier()

        pltpu.sync_copy(agg_sp.at[pl.ds(sid * RPT, RPT)],
                        out_hbm.at[cid, pl.ds(sid * RPT, RPT)])

        @pl.when(sid == NS - 1)
        def _():
            pltpu.sync_copy(agg_sp.at[pl.ds(NS * RPT, N - NS * RPT)],
                            out_hbm.at[cid, pl.ds(NS * RPT, N - NS * RPT)])


def _edge_phase(w, haug, alsrc, aldst, bnd, esrc, edst):
    mesh = plsc.VectorSubcoreMesh(core_axis_name="c", subcore_axis_name="s",
                                  num_cores=NC, num_subcores=NS)
    fn = pl.kernel(
        functools.partial(_edge_body, w),
        out_type=jax.ShapeDtypeStruct((NC, N, w), jnp.float32),
        mesh=mesh,
        scratch_types=[
            pltpu.VMEM((NCHUNK, CH), jnp.int32),
            pltpu.VMEM((NCHUNK, CH), jnp.int32),
            pltpu.VMEM((N,), jnp.float32),
            pltpu.VMEM((N,), jnp.float32),
            pltpu.VMEM((L,), jnp.float32),
            pltpu.VMEM((CH,), jnp.float32),
            pltpu.VMEM((CH, w), jnp.float32),
            pltpu.VMEM((ZCH, w), jnp.float32),
            pltpu.VMEM_SHARED((N, w), jnp.float32),
            pltpu.SemaphoreType.DMA,
        ],
        compiler_params=pltpu.CompilerParams(needs_layout_passes=False,
                                             use_tc_tiling_on_sc=False),
        name=f"gat_edge_w{w}",
    )
    return fn(haug, alsrc, aldst, bnd, esrc, edst)


def kernel(x, edge_index, W1, a_src1, a_dst1, b1, W2, a_src2, a_dst2, b2,
           Wlin, blin):
    R = 400
    G = N // R

    a2_1 = jnp.stack([a_src1.reshape(HID), a_dst1.reshape(HID)], axis=1)
    a2_2 = jnp.stack([a_src2.reshape(OUT), a_dst2.reshape(OUT)], axis=1)

    haug, alsrc1, aldst1, bnd1 = pl.pallas_call(
        _proj1_body,
        grid=(G,),
        in_specs=[
            pl.BlockSpec((R, IN), lambda i: (i, 0)),
            pl.BlockSpec((IN, HID), lambda i: (0, 0)),
            pl.BlockSpec((HID, 2), lambda i: (0, 0)),
        ],
        out_specs=[
            pl.BlockSpec((R, 80), lambda i: (i, 0)),
            pl.BlockSpec((R, 1), lambda i: (i, 0)),
            pl.BlockSpec((R, 1), lambda i: (i, 0)),
            pl.BlockSpec((1, 16), lambda i: (0, 0)),
        ],
        out_shape=[
            jax.ShapeDtypeStruct((N, 80), jnp.float32),
            jax.ShapeDtypeStruct((N, 1), jnp.float32),
            jax.ShapeDtypeStruct((N, 1), jnp.float32),
            jax.ShapeDtypeStruct((1, 16), jnp.float32),
        ],
    )(x, W1, a2_1)

    esrc = edge_index[0]
    edst = edge_index[1]
    agg1 = _edge_phase(80, haug, alsrc1.reshape(N), aldst1.reshape(N),
                       bnd1.reshape(16), esrc, edst)

    h2aug, alsrc2, aldst2, bnd2 = pl.pallas_call(
        _combine1_body,
        grid=(G,),
        in_specs=[
            pl.BlockSpec((NC, R, 80), lambda i: (0, i, 0)),
            pl.BlockSpec((1, HID), lambda i: (0, 0)),
            pl.BlockSpec((HID, OUT), lambda i: (0, 0)),
            pl.BlockSpec((OUT, 2), lambda i: (0, 0)),
        ],
        out_specs=[
            pl.BlockSpec((R, 48), lambda i: (i, 0)),
            pl.BlockSpec((R, 1), lambda i: (i, 0)),
            pl.BlockSpec((R, 1), lambda i: (i, 0)),
            pl.BlockSpec((1, 16), lambda i: (0, 0)),
        ],
        out_shape=[
            jax.ShapeDtypeStruct((N, 48), jnp.float32),
            jax.ShapeDtypeStruct((N, 1), jnp.float32),
            jax.ShapeDtypeStruct((N, 1), jnp.float32),
            jax.ShapeDtypeStruct((1, 16), jnp.float32),
        ],
    )(agg1, b1.reshape(1, HID), W2, a2_2)

    agg2 = _edge_phase(48, h2aug, alsrc2.reshape(N), aldst2.reshape(N),
                       bnd2.reshape(16), esrc, edst)

    z, x_ = pl.pallas_call(
        _head_body,
        grid=(G,),
        in_specs=[
            pl.BlockSpec((NC, R, 48), lambda i: (0, i, 0)),
            pl.BlockSpec((1, OUT), lambda i: (0, 0)),
            pl.BlockSpec((OUT, IN), lambda i: (0, 0)),
            pl.BlockSpec((1, IN), lambda i: (0, 0)),
        ],
        out_specs=[
            pl.BlockSpec((R, OUT), lambda i: (i, 0)),
            pl.BlockSpec((R, IN), lambda i: (i, 0)),
        ],
        out_shape=[
            jax.ShapeDtypeStruct((N, OUT), jnp.float32),
            jax.ShapeDtypeStruct((N, IN), jnp.float32),
        ],
    )(agg2, b2.reshape(1, OUT), Wlin, blin.reshape(1, IN))

    B = 512
    GA = pl.cdiv(N, B)
    zp = jnp.pad(z, ((0, GA * B - N), (0, 0)))
    a_pred = pl.pallas_call(
        _adj_body,
        grid=(GA, GA),
        in_specs=[
            pl.BlockSpec((B, OUT), lambda i, j: (i, 0)),
            pl.BlockSpec((B, OUT), lambda i, j: (j, 0)),
        ],
        out_specs=pl.BlockSpec((B, B), lambda i, j: (i, j)),
        out_shape=jax.ShapeDtypeStruct((N, N), jnp.float32),
    )(zp, zp)

    return (a_pred, z, x_)


# speedup vs baseline: 21.9018x; 1.2036x over previous
import functools

import jax
import jax.numpy as jnp
from jax import lax
from jax.experimental import pallas as pl
from jax.experimental.pallas import tpu as pltpu
from jax.experimental.pallas import tpu_sc as plsc

N = 10000
E = 320000
IN = 128
HID = 64
OUT = 32

NC = 2
NS = 16
L = 16
NW = NC * NS
EPW = E // NW
CH = 80
NCHUNK = EPW // CH
RPT = 624
ZCH = 16


def _proj1_body(x_ref, w_ref, a2_ref, haug_ref, alsrc_ref, aldst_ref,
                bnd_ref):
    h = jnp.dot(x_ref[...], w_ref[...], preferred_element_type=jnp.float32)
    r = h.shape[0]
    ones = jnp.ones((r, 1), jnp.float32)
    pad = jnp.zeros((r, 80 - HID - 1), jnp.float32)
    haug_ref[...] = jnp.concatenate([h, ones, pad], axis=1)
    als = jnp.dot(h, a2_ref[...], preferred_element_type=jnp.float32)
    alsrc_ref[...] = als[:, 0:1]
    aldst_ref[...] = als[:, 1:2]
    _accum_bound(als, bnd_ref)


def _accum_bound(als, bnd_ref):
    @pl.when(pl.program_id(0) == 0)
    def _():
        bnd_ref[...] = jnp.full((1, 16), -3e38, jnp.float32)
    m = jnp.max(als[:, 0])
    d = jnp.max(als[:, 1])
    iot = lax.broadcasted_iota(jnp.int32, (1, 16), 1)
    upd = jnp.where(iot == 0, m, jnp.where(iot == 1, d, -3e38))
    bnd_ref[...] = jnp.maximum(bnd_ref[...], upd)


def _combine1_body(agg_ref, b1_ref, w2_ref, a2_ref, h2aug_ref, alsrc_ref,
                   aldst_ref, bnd_ref):
    p = agg_ref[...]
    s = p[0] + p[1]
    den = s[:, HID:HID + 1]
    h1 = s[:, :HID] / (den + 1e-16) + b1_ref[...]
    h2 = jnp.dot(h1, w2_ref[...], preferred_element_type=jnp.float32)
    r = h2.shape[0]
    ones = jnp.ones((r, 1), jnp.float32)
    pad = jnp.zeros((r, 48 - OUT - 1), jnp.float32)
    h2aug_ref[...] = jnp.concatenate([h2, ones, pad], axis=1)
    als = jnp.dot(h2, a2_ref[...], preferred_element_type=jnp.float32)
    alsrc_ref[...] = als[:, 0:1]
    aldst_ref[...] = als[:, 1:2]
    _accum_bound(als, bnd_ref)


def _head_body(agg_ref, b2_ref, wlin_ref, blin_ref, z_ref, xr_ref):
    p = agg_ref[...]
    s = p[0] + p[1]
    den = s[:, OUT:OUT + 1]
    h = s[:, :OUT] / (den + 1e-16) + b2_ref[...]
    nrm = jnp.sqrt(jnp.sum(h * h, axis=1, keepdims=True))
    z = h / jnp.maximum(nrm, 1e-12)
    z_ref[...] = z
    zw = jnp.dot(z, wlin_ref[...], preferred_element_type=jnp.float32)
    zw = zw + blin_ref[...]
    xr_ref[...] = jnp.where(zw > 0, zw, jnp.exp(zw) - 1.0)


def _adj_body(zi_ref, zj_ref, out_ref):
    g = lax.dot_general(zi_ref[...], zj_ref[...], (((1,), (1,)), ((), ())),
                        preferred_element_type=jnp.float32)
    out_ref[...] = 1.0 / (1.0 + jnp.exp(-g))


def _edge_body(w, haug_hbm, alsrc_hbm, aldst_hbm, bnd_hbm, esrc_hbm, edst_hbm,
               out_hbm, src2d, dst2d, alsrc_v, aldst_v, bndv, exbuf, rows_v,
               zbuf, agg_sp, sem):
    cid = lax.axis_index("c")
    sid = lax.axis_index("s")
    wid = cid * NS + sid
    base = wid * EPW

    def zero_zbuf(i, _):
        for k in range(w // L):
            zbuf[i, pl.ds(k * L, L)] = jnp.zeros((L,), jnp.float32)
        return 0
    lax.fori_loop(0, ZCH, zero_zbuf, 0)

    if True:
        def zero_agg(i, _):
            pltpu.sync_copy(zbuf, agg_sp.at[pl.ds(sid * RPT + i * ZCH, ZCH)])
            return 0
        lax.fori_loop(0, RPT // ZCH, zero_agg, 0)

        @pl.when(sid == NS - 1)
        def _():
            pltpu.sync_copy(zbuf, agg_sp.at[pl.ds(NS * RPT, N - NS * RPT)])

        def load_idx(i, _):
            pltpu.sync_copy(esrc_hbm.at[pl.ds(base + i * CH, CH)],
                            src2d.at[i])
            pltpu.sync_copy(edst_hbm.at[pl.ds(base + i * CH, CH)],
                            dst2d.at[i])
            return 0
        lax.fori_loop(0, NCHUNK, load_idx, 0)

        pltpu.sync_copy(alsrc_hbm, alsrc_v)
        pltpu.sync_copy(aldst_hbm, aldst_v)
        pltpu.sync_copy(bnd_hbm, bndv)
        bv = bndv[...]
        c = bv[0] + bv[1]
        bound = jnp.where(c > 0, c, 0.2 * c)

        plsc.subcore_barrier()

        def chunk(j, _):
            for k in range(CH // L):
                si = src2d[j, pl.ds(k * L, L)]
                di = dst2d[j, pl.ds(k * L, L)]
                sv = plsc.load_gather(alsrc_v, [si])
                dv = plsc.load_gather(aldst_v, [di])
                e = sv + dv
                e = jnp.where(e > 0, e, 0.2 * e)
                exbuf[pl.ds(k * L, L)] = jnp.exp(e - bound)

            pltpu.async_copy(haug_hbm.at[src2d.at[j]], rows_v, sem).wait()

            def scale(g, _):
                exv = exbuf[pl.ds(g * L, L)]
                for l in range(L):
                    s = exv[l]
                    ei = g * L + l
                    for k in range(w // L):
                        rows_v[ei, pl.ds(k * L, L)] = (
                            rows_v[ei, pl.ds(k * L, L)] * s)
                return 0
            lax.fori_loop(0, CH // L, scale, 0)

            pltpu.sync_copy(rows_v, agg_sp.at[dst2d.at[j]], add=True)
            return 0

        lax.fori_loop(0, NCHUNK, chunk, 0)

        plsc.subcore_barrier()

        pltpu.sync_copy(agg_sp.at[pl.ds(sid * RPT, RPT)],
                        out_hbm.at[cid, pl.ds(sid * RPT, RPT)])

        @pl.when(sid == NS - 1)
        def _():
            pltpu.sync_copy(agg_sp.at[pl.ds(NS * RPT, N - NS * RPT)],
                            out_hbm.at[cid, pl.ds(NS * RPT, N - NS * RPT)])


def _edge_phase(w, haug, alsrc, aldst, bnd, esrc, edst):
    mesh = plsc.VectorSubcoreMesh(core_axis_name="c", subcore_axis_name="s",
                                  num_cores=NC, num_subcores=NS)
    fn = pl.kernel(
        functools.partial(_edge_body, w),
        out_type=jax.ShapeDtypeStruct((NC, N, w), jnp.float32),
        mesh=mesh,
        scratch_types=[
            pltpu.VMEM((NCHUNK, CH), jnp.int32),
            pltpu.VMEM((NCHUNK, CH), jnp.int32),
            pltpu.VMEM((N,), jnp.float32),
            pltpu.VMEM((N,), jnp.float32),
            pltpu.VMEM((L,), jnp.float32),
            pltpu.VMEM((CH,), jnp.float32),
            pltpu.VMEM((CH, w), jnp.float32),
            pltpu.VMEM((ZCH, w), jnp.float32),
            pltpu.VMEM_SHARED((N, w), jnp.float32),
            pltpu.SemaphoreType.DMA,
        ],
        compiler_params=pltpu.CompilerParams(needs_layout_passes=False,
                                             use_tc_tiling_on_sc=False),
        name=f"gat_edge_w{w}",
    )
    return fn(haug, alsrc, aldst, bnd, esrc, edst)


def kernel(x, edge_index, W1, a_src1, a_dst1, b1, W2, a_src2, a_dst2, b2,
           Wlin, blin):
    R = 400
    G = N // R

    a2_1 = jnp.stack([a_src1.reshape(HID), a_dst1.reshape(HID)], axis=1)
    a2_2 = jnp.stack([a_src2.reshape(OUT), a_dst2.reshape(OUT)], axis=1)

    haug, alsrc1, aldst1, bnd1 = pl.pallas_call(
        _proj1_body,
        grid=(G,),
        in_specs=[
            pl.BlockSpec((R, IN), lambda i: (i, 0)),
            pl.BlockSpec((IN, HID), lambda i: (0, 0)),
            pl.BlockSpec((HID, 2), lambda i: (0, 0)),
        ],
        out_specs=[
            pl.BlockSpec((R, 80), lambda i: (i, 0)),
            pl.BlockSpec((R, 1), lambda i: (i, 0)),
            pl.BlockSpec((R, 1), lambda i: (i, 0)),
            pl.BlockSpec((1, 16), lambda i: (0, 0)),
        ],
        out_shape=[
            jax.ShapeDtypeStruct((N, 80), jnp.float32),
            jax.ShapeDtypeStruct((N, 1), jnp.float32),
            jax.ShapeDtypeStruct((N, 1), jnp.float32),
            jax.ShapeDtypeStruct((1, 16), jnp.float32),
        ],
    )(x, W1, a2_1)

    esrc = edge_index[0]
    edst = edge_index[1]
    agg1 = _edge_phase(80, haug, alsrc1.reshape(N), aldst1.reshape(N),
                       bnd1.reshape(16), esrc, edst)

    h2aug, alsrc2, aldst2, bnd2 = pl.pallas_call(
        _combine1_body,
        grid=(G,),
        in_specs=[
            pl.BlockSpec((NC, R, 80), lambda i: (0, i, 0)),
            pl.BlockSpec((1, HID), lambda i: (0, 0)),
            pl.BlockSpec((HID, OUT), lambda i: (0, 0)),
            pl.BlockSpec((OUT, 2), lambda i: (0, 0)),
        ],
        out_specs=[
            pl.BlockSpec((R, 48), lambda i: (i, 0)),
            pl.BlockSpec((R, 1), lambda i: (i, 0)),
            pl.BlockSpec((R, 1), lambda i: (i, 0)),
            pl.BlockSpec((1, 16), lambda i: (0, 0)),
        ],
        out_shape=[
            jax.ShapeDtypeStruct((N, 48), jnp.float32),
            jax.ShapeDtypeStruct((N, 1), jnp.float32),
            jax.ShapeDtypeStruct((N, 1), jnp.float32),
            jax.ShapeDtypeStruct((1, 16), jnp.float32),
        ],
    )(agg1, b1.reshape(1, HID), W2, a2_2)

    agg2 = _edge_phase(48, h2aug, alsrc2.reshape(N), aldst2.reshape(N),
                       bnd2.reshape(16), esrc, edst)

    z, x_ = pl.pallas_call(
        _head_body,
        grid=(G,),
        in_specs=[
            pl.BlockSpec((NC, R, 48), lambda i: (0, i, 0)),
            pl.BlockSpec((1, OUT), lambda i: (0, 0)),
            pl.BlockSpec((OUT, IN), lambda i: (0, 0)),
            pl.BlockSpec((1, IN), lambda i: (0, 0)),
        ],
        out_specs=[
            pl.BlockSpec((R, OUT), lambda i: (i, 0)),
            pl.BlockSpec((R, IN), lambda i: (i, 0)),
        ],
        out_shape=[
            jax.ShapeDtypeStruct((N, OUT), jnp.float32),
            jax.ShapeDtypeStruct((N, IN), jnp.float32),
        ],
    )(agg2, b2.reshape(1, OUT), Wlin, blin.reshape(1, IN))

    B = 1024
    GA = pl.cdiv(N, B)
    zp = jnp.pad(z, ((0, GA * B - N), (0, 0)))
    a_pred = pl.pallas_call(
        _adj_body,
        grid=(GA, GA),
        in_specs=[
            pl.BlockSpec((B, OUT), lambda i, j: (i, 0)),
            pl.BlockSpec((B, OUT), lambda i, j: (j, 0)),
        ],
        out_specs=pl.BlockSpec((B, B), lambda i, j: (i, j)),
        out_shape=jax.ShapeDtypeStruct((N, N), jnp.float32),
    )(zp, zp)

    return (a_pred, z, x_)


# speedup vs baseline: 22.9319x; 1.0470x over previous
import functools

import jax
import jax.numpy as jnp
from jax import lax
from jax.experimental import pallas as pl
from jax.experimental.pallas import tpu as pltpu
from jax.experimental.pallas import tpu_sc as plsc

N = 10000
E = 320000
IN = 128
HID = 64
OUT = 32

NC = 2
NS = 16
L = 16
NW = NC * NS
EPW = E // NW
CH = 80
NCHUNK = EPW // CH
RPT = 624
ZCH = 16


def _proj1_body(x_ref, w_ref, a2_ref, haug_ref, alsrc_ref, aldst_ref,
                bnd_ref):
    h = jnp.dot(x_ref[...], w_ref[...], preferred_element_type=jnp.float32)
    r = h.shape[0]
    ones = jnp.ones((r, 1), jnp.float32)
    pad = jnp.zeros((r, 80 - HID - 1), jnp.float32)
    haug_ref[...] = jnp.concatenate([h, ones, pad], axis=1)
    als = jnp.dot(h, a2_ref[...], preferred_element_type=jnp.float32)
    alsrc_ref[...] = als[:, 0:1]
    aldst_ref[...] = als[:, 1:2]
    _accum_bound(als, bnd_ref)


def _accum_bound(als, bnd_ref):
    @pl.when(pl.program_id(0) == 0)
    def _():
        bnd_ref[...] = jnp.full((1, 16), -3e38, jnp.float32)
    m = jnp.max(als[:, 0])
    d = jnp.max(als[:, 1])
    iot = lax.broadcasted_iota(jnp.int32, (1, 16), 1)
    upd = jnp.where(iot == 0, m, jnp.where(iot == 1, d, -3e38))
    bnd_ref[...] = jnp.maximum(bnd_ref[...], upd)


def _combine1_body(agg_ref, b1_ref, w2_ref, a2_ref, h2aug_ref, alsrc_ref,
                   aldst_ref, bnd_ref):
    p = agg_ref[...]
    s = p[0] + p[1]
    den = s[:, HID:HID + 1]
    h1 = s[:, :HID] / (den + 1e-16) + b1_ref[...]
    h2 = jnp.dot(h1, w2_ref[...], preferred_element_type=jnp.float32)
    r = h2.shape[0]
    ones = jnp.ones((r, 1), jnp.float32)
    pad = jnp.zeros((r, 48 - OUT - 1), jnp.float32)
    h2aug_ref[...] = jnp.concatenate([h2, ones, pad], axis=1)
    als = jnp.dot(h2, a2_ref[...], preferred_element_type=jnp.float32)
    alsrc_ref[...] = als[:, 0:1]
    aldst_ref[...] = als[:, 1:2]
    _accum_bound(als, bnd_ref)


def _head_body(agg_ref, b2_ref, wlin_ref, blin_ref, z_ref, xr_ref):
    p = agg_ref[...]
    s = p[0] + p[1]
    den = s[:, OUT:OUT + 1]
    h = s[:, :OUT] / (den + 1e-16) + b2_ref[...]
    nrm = jnp.sqrt(jnp.sum(h * h, axis=1, keepdims=True))
    z = h / jnp.maximum(nrm, 1e-12)
    z_ref[...] = z
    zw = jnp.dot(z, wlin_ref[...], preferred_element_type=jnp.float32)
    zw = zw + blin_ref[...]
    xr_ref[...] = jnp.where(zw > 0, zw, jnp.exp(zw) - 1.0)


def _adj_body(zi_ref, zj_ref, out_ref):
    g = lax.dot_general(zi_ref[...], zj_ref[...], (((1,), (1,)), ((), ())),
                        preferred_element_type=jnp.float32)
    out_ref[...] = 1.0 / (1.0 + jnp.exp(-g))


def _edge_body(w, haug_hbm, alsrc_hbm, aldst_hbm, bnd_hbm, esrc_hbm, edst_hbm,
               out_hbm, src2d, dst2d, alsrc_v, aldst_v, bndv, exbuf, rows_v,
               zbuf, agg_sp, sem):
    cid = lax.axis_index("c")
    sid = lax.axis_index("s")
    wid = cid * NS + sid
    base = wid * EPW

    def zero_zbuf(i, _):
        for k in range(w // L):
            zbuf[i, pl.ds(k * L, L)] = jnp.zeros((L,), jnp.float32)
        return 0
    lax.fori_loop(0, ZCH, zero_zbuf, 0)

    if True:
        def zero_agg(i, _):
            pltpu.sync_copy(zbuf, agg_sp.at[pl.ds(sid * RPT + i * ZCH, ZCH)])
            return 0
        lax.fori_loop(0, RPT // ZCH, zero_agg, 0)

        @pl.when(sid == NS - 1)
        def _():
            pltpu.sync_copy(zbuf, agg_sp.at[pl.ds(NS * RPT, N - NS * RPT)])

        def load_idx(i, _):
            pltpu.sync_copy(esrc_hbm.at[pl.ds(base + i * CH, CH)],
                            src2d.at[i])
            pltpu.sync_copy(edst_hbm.at[pl.ds(base + i * CH, CH)],
                            dst2d.at[i])
            return 0
        lax.fori_loop(0, NCHUNK, load_idx, 0)

        pltpu.sync_copy(alsrc_hbm, alsrc_v)
        pltpu.sync_copy(aldst_hbm, aldst_v)
        pltpu.sync_copy(bnd_hbm, bndv)
        bv = bndv[...]
        c = bv[0] + bv[1]
        bound = jnp.where(c > 0, c, 0.2 * c)

        plsc.subcore_barrier()

        def chunk(j, _):
            for k in range(CH // L):
                si = src2d[j, pl.ds(k * L, L)]
                di = dst2d[j, pl.ds(k * L, L)]
                sv = plsc.load_gather(alsrc_v, [si])
                dv = plsc.load_gather(aldst_v, [di])
                e = sv + dv
                e = jnp.where(e > 0, e, 0.2 * e)
                exbuf[pl.ds(k * L, L)] = jnp.exp(e - bound)

            pltpu.async_copy(haug_hbm.at[src2d.at[j]], rows_v, sem).wait()

            def scale(g, _):
                exv = exbuf[pl.ds(g * L, L)]
                for l in range(L):
                    s = exv[l]
                    ei = g * L + l
                    for k in range(w // L):
                        rows_v[ei, pl.ds(k * L, L)] = (
                            rows_v[ei, pl.ds(k * L, L)] * s)
                return 0
            lax.fori_loop(0, CH // L, scale, 0)

            pltpu.sync_copy(rows_v, agg_sp.at[dst2d.at[j]], add=True)
            return 0

        lax.fori_loop(0, NCHUNK, chunk, 0)

        plsc.subcore_barrier()

        pltpu.sync_copy(agg_sp.at[pl.ds(sid * RPT, RPT)],
                        out_hbm.at[cid, pl.ds(sid * RPT, RPT)])

        @pl.when(sid == NS - 1)
        def _():
            pltpu.sync_copy(agg_sp.at[pl.ds(NS * RPT, N - NS * RPT)],
                            out_hbm.at[cid, pl.ds(NS * RPT, N - NS * RPT)])


def _edge_phase(w, haug, alsrc, aldst, bnd, esrc, edst):
    mesh = plsc.VectorSubcoreMesh(core_axis_name="c", subcore_axis_name="s",
                                  num_cores=NC, num_subcores=NS)
    fn = pl.kernel(
        functools.partial(_edge_body, w),
        out_type=jax.ShapeDtypeStruct((NC, N, w), jnp.float32),
        mesh=mesh,
        scratch_types=[
            pltpu.VMEM((NCHUNK, CH), jnp.int32),
            pltpu.VMEM((NCHUNK, CH), jnp.int32),
            pltpu.VMEM((N,), jnp.float32),
            pltpu.VMEM((N,), jnp.float32),
            pltpu.VMEM((L,), jnp.float32),
            pltpu.VMEM((CH,), jnp.float32),
            pltpu.VMEM((CH, w), jnp.float32),
            pltpu.VMEM((ZCH, w), jnp.float32),
            pltpu.VMEM_SHARED((N, w), jnp.float32),
            pltpu.SemaphoreType.DMA,
        ],
        compiler_params=pltpu.CompilerParams(needs_layout_passes=False,
                                             use_tc_tiling_on_sc=False),
        name=f"gat_edge_w{w}",
    )
    return fn(haug, alsrc, aldst, bnd, esrc, edst)


def kernel(x, edge_index, W1, a_src1, a_dst1, b1, W2, a_src2, a_dst2, b2,
           Wlin, blin):
    R = 400
    G = N // R

    a2_1 = jnp.stack([a_src1.reshape(HID), a_dst1.reshape(HID)], axis=1)
    a2_2 = jnp.stack([a_src2.reshape(OUT), a_dst2.reshape(OUT)], axis=1)

    haug, alsrc1, aldst1, bnd1 = pl.pallas_call(
        _proj1_body,
        grid=(G,),
        in_specs=[
            pl.BlockSpec((R, IN), lambda i: (i, 0)),
            pl.BlockSpec((IN, HID), lambda i: (0, 0)),
            pl.BlockSpec((HID, 2), lambda i: (0, 0)),
        ],
        out_specs=[
            pl.BlockSpec((R, 80), lambda i: (i, 0)),
            pl.BlockSpec((R, 1), lambda i: (i, 0)),
            pl.BlockSpec((R, 1), lambda i: (i, 0)),
            pl.BlockSpec((1, 16), lambda i: (0, 0)),
        ],
        out_shape=[
            jax.ShapeDtypeStruct((N, 80), jnp.float32),
            jax.ShapeDtypeStruct((N, 1), jnp.float32),
            jax.ShapeDtypeStruct((N, 1), jnp.float32),
            jax.ShapeDtypeStruct((1, 16), jnp.float32),
        ],
    )(x, W1, a2_1)

    esrc = edge_index[0]
    edst = edge_index[1]
    agg1 = _edge_phase(80, haug, alsrc1.reshape(N), aldst1.reshape(N),
                       bnd1.reshape(16), esrc, edst)

    h2aug, alsrc2, aldst2, bnd2 = pl.pallas_call(
        _combine1_body,
        grid=(G,),
        in_specs=[
            pl.BlockSpec((NC, R, 80), lambda i: (0, i, 0)),
            pl.BlockSpec((1, HID), lambda i: (0, 0)),
            pl.BlockSpec((HID, OUT), lambda i: (0, 0)),
            pl.BlockSpec((OUT, 2), lambda i: (0, 0)),
        ],
        out_specs=[
            pl.BlockSpec((R, 48), lambda i: (i, 0)),
            pl.BlockSpec((R, 1), lambda i: (i, 0)),
            pl.BlockSpec((R, 1), lambda i: (i, 0)),
            pl.BlockSpec((1, 16), lambda i: (0, 0)),
        ],
        out_shape=[
            jax.ShapeDtypeStruct((N, 48), jnp.float32),
            jax.ShapeDtypeStruct((N, 1), jnp.float32),
            jax.ShapeDtypeStruct((N, 1), jnp.float32),
            jax.ShapeDtypeStruct((1, 16), jnp.float32),
        ],
    )(agg1, b1.reshape(1, HID), W2, a2_2)

    agg2 = _edge_phase(48, h2aug, alsrc2.reshape(N), aldst2.reshape(N),
                       bnd2.reshape(16), esrc, edst)

    z, x_ = pl.pallas_call(
        _head_body,
        grid=(G,),
        in_specs=[
            pl.BlockSpec((NC, R, 48), lambda i: (0, i, 0)),
            pl.BlockSpec((1, OUT), lambda i: (0, 0)),
            pl.BlockSpec((OUT, IN), lambda i: (0, 0)),
            pl.BlockSpec((1, IN), lambda i: (0, 0)),
        ],
        out_specs=[
            pl.BlockSpec((R, OUT), lambda i: (i, 0)),
            pl.BlockSpec((R, IN), lambda i: (i, 0)),
        ],
        out_shape=[
            jax.ShapeDtypeStruct((N, OUT), jnp.float32),
            jax.ShapeDtypeStruct((N, IN), jnp.float32),
        ],
    )(agg2, b2.reshape(1, OUT), Wlin, blin.reshape(1, IN))

    B = 2048
    GA = pl.cdiv(N, B)
    zp = jnp.pad(z, ((0, GA * B - N), (0, 0)))
    a_pred = pl.pallas_call(
        _adj_body,
        grid=(GA, GA),
        in_specs=[
            pl.BlockSpec((B, OUT), lambda i, j: (i, 0)),
            pl.BlockSpec((B, OUT), lambda i, j: (j, 0)),
        ],
        out_specs=pl.BlockSpec((B, B), lambda i, j: (i, j)),
        out_shape=jax.ShapeDtypeStruct((N, N), jnp.float32),
    )(zp, zp)

    return (a_pred, z, x_)


# speedup vs baseline: 37.0527x; 1.6158x over previous
import functools

import jax
import jax.numpy as jnp
from jax import lax
from jax.experimental import pallas as pl
from jax.experimental.pallas import tpu as pltpu
from jax.experimental.pallas import tpu_sc as plsc

N = 10000
E = 320000
IN = 128
HID = 64
OUT = 32

NC = 2
NS = 16
L = 16
NW = NC * NS
EPW = E // NW
CH = 80
NCHUNK = EPW // CH
RPT = 624
ZCH = 208


def _proj1_body(x_ref, w_ref, a2_ref, haug_ref, alsrc_ref, aldst_ref,
                bnd_ref):
    h = jnp.dot(x_ref[...], w_ref[...], preferred_element_type=jnp.float32)
    r = h.shape[0]
    ones = jnp.ones((r, 1), jnp.float32)
    pad = jnp.zeros((r, 80 - HID - 1), jnp.float32)
    haug_ref[...] = jnp.concatenate([h, ones, pad], axis=1)
    als = jnp.dot(h, a2_ref[...], preferred_element_type=jnp.float32)
    alsrc_ref[...] = als[:, 0:1]
    aldst_ref[...] = als[:, 1:2]
    _accum_bound(als, bnd_ref)


def _accum_bound(als, bnd_ref):
    @pl.when(pl.program_id(0) == 0)
    def _():
        bnd_ref[...] = jnp.full((1, 16), -3e38, jnp.float32)
    m = jnp.max(als[:, 0])
    d = jnp.max(als[:, 1])
    iot = lax.broadcasted_iota(jnp.int32, (1, 16), 1)
    upd = jnp.where(iot == 0, m, jnp.where(iot == 1, d, -3e38))
    bnd_ref[...] = jnp.maximum(bnd_ref[...], upd)


def _combine1_body(agg_ref, b1_ref, w2_ref, a2_ref, h2aug_ref, alsrc_ref,
                   aldst_ref, bnd_ref):
    p = agg_ref[...]
    s = p[0] + p[1]
    den = s[:, HID:HID + 1]
    h1 = s[:, :HID] / (den + 1e-16) + b1_ref[...]
    h2 = jnp.dot(h1, w2_ref[...], preferred_element_type=jnp.float32)
    r = h2.shape[0]
    ones = jnp.ones((r, 1), jnp.float32)
    pad = jnp.zeros((r, 48 - OUT - 1), jnp.float32)
    h2aug_ref[...] = jnp.concatenate([h2, ones, pad], axis=1)
    als = jnp.dot(h2, a2_ref[...], preferred_element_type=jnp.float32)
    alsrc_ref[...] = als[:, 0:1]
    aldst_ref[...] = als[:, 1:2]
    _accum_bound(als, bnd_ref)


def _head_body(agg_ref, b2_ref, wlin_ref, blin_ref, z_ref, xr_ref):
    p = agg_ref[...]
    s = p[0] + p[1]
    den = s[:, OUT:OUT + 1]
    h = s[:, :OUT] / (den + 1e-16) + b2_ref[...]
    nrm = jnp.sqrt(jnp.sum(h * h, axis=1, keepdims=True))
    z = h / jnp.maximum(nrm, 1e-12)
    z_ref[...] = z
    zw = jnp.dot(z, wlin_ref[...], preferred_element_type=jnp.float32)
    zw = zw + blin_ref[...]
    xr_ref[...] = jnp.where(zw > 0, zw, jnp.exp(zw) - 1.0)


def _adj_body(zi_ref, zj_ref, out_ref):
    g = lax.dot_general(zi_ref[...], zj_ref[...], (((1,), (1,)), ((), ())),
                        preferred_element_type=jnp.float32)
    out_ref[...] = 1.0 / (1.0 + jnp.exp(-g))


def _edge_body(w, haug_hbm, alsrc_hbm, aldst_hbm, bnd_hbm, esrc_hbm, edst_hbm,
               out_hbm, src2d, dst2d, alsrc_v, aldst_v, bndv, exbuf, rows_a,
               rows_b, zbuf, agg_sp, sem_a, sem_b):
    cid = lax.axis_index("c")
    sid = lax.axis_index("s")
    wid = cid * NS + sid

    def zero_zbuf(i, _):
        for k in range(w // L):
            zbuf[i, pl.ds(k * L, L)] = jnp.zeros((L,), jnp.float32)
        return 0
    lax.fori_loop(0, ZCH, zero_zbuf, 0)

    for i in range(RPT // ZCH):
        pltpu.sync_copy(zbuf, agg_sp.at[pl.ds(sid * RPT + i * ZCH, ZCH)])

    @pl.when(sid == NS - 1)
    def _():
        pltpu.sync_copy(zbuf.at[pl.ds(0, N - NS * RPT)],
                        agg_sp.at[pl.ds(NS * RPT, N - NS * RPT)])

    pltpu.sync_copy(esrc_hbm.at[wid], src2d)
    pltpu.sync_copy(edst_hbm.at[wid], dst2d)
    pltpu.sync_copy(alsrc_hbm, alsrc_v)
    pltpu.sync_copy(aldst_hbm, aldst_v)
    pltpu.sync_copy(bnd_hbm, bndv)
    bv = bndv[...]
    c = bv[0] + bv[1]
    bound = jnp.where(c > 0, c, 0.2 * c)

    plsc.subcore_barrier()

    def compute_ex(j):
        for k in range(CH // L):
            si = src2d[j, pl.ds(k * L, L)]
            di = dst2d[j, pl.ds(k * L, L)]
            sv = plsc.load_gather(alsrc_v, [si])
            dv = plsc.load_gather(aldst_v, [di])
            e = sv + dv
            e = jnp.where(e > 0, e, 0.2 * e)
            exbuf[pl.ds(k * L, L)] = jnp.exp(e - bound)

    def gather(j, rv, sem):
        pltpu.async_copy(haug_hbm.at[src2d.at[j]], rv, sem)

    def wait_gather(rv, sem):
        pltpu.make_async_copy(haug_hbm.at[src2d.at[0]], rv, sem).wait()

    def scale(rv):
        def sg(g, _):
            exv = exbuf[pl.ds(g * L, L)]
            for l in range(L):
                s = exv[l]
                ei = g * L + l
                for k in range(w // L):
                    rv[ei, pl.ds(k * L, L)] = rv[ei, pl.ds(k * L, L)] * s
            return 0
        lax.fori_loop(0, CH // L, sg, 0)

    def process(j, rv, sem):
        compute_ex(j)
        wait_gather(rv, sem)
        scale(rv)

    gather(0, rows_a, sem_a)

    def pair(i, _):
        j0 = 2 * i
        compute_ex(j0)
        wait_gather(rows_a, sem_a)
        gather(j0 + 1, rows_b, sem_b)
        scale(rows_a)
        pltpu.sync_copy(rows_a, agg_sp.at[dst2d.at[j0]], add=True)
        compute_ex(j0 + 1)
        wait_gather(rows_b, sem_b)
        gather(j0 + 2, rows_a, sem_a)
        scale(rows_b)
        pltpu.sync_copy(rows_b, agg_sp.at[dst2d.at[j0 + 1]], add=True)
        return 0

    lax.fori_loop(0, (NCHUNK - 1) // 2, pair, 0)

    process(NCHUNK - 1, rows_a, sem_a)
    pltpu.sync_copy(rows_a, agg_sp.at[dst2d.at[NCHUNK - 1]], add=True)

    plsc.subcore_barrier()

    pltpu.sync_copy(agg_sp.at[pl.ds(sid * RPT, RPT)],
                    out_hbm.at[cid, pl.ds(sid * RPT, RPT)])

    @pl.when(sid == NS - 1)
    def _():
        pltpu.sync_copy(agg_sp.at[pl.ds(NS * RPT, N - NS * RPT)],
                        out_hbm.at[cid, pl.ds(NS * RPT, N - NS * RPT)])


def _edge_phase(w, haug, alsrc, aldst, bnd, esrc, edst):
    mesh = plsc.VectorSubcoreMesh(core_axis_name="c", subcore_axis_name="s",
                                  num_cores=NC, num_subcores=NS)
    fn = pl.kernel(
        functools.partial(_edge_body, w),
        out_type=jax.ShapeDtypeStruct((NC, N, w), jnp.float32),
        mesh=mesh,
        scratch_types=[
            pltpu.VMEM((NCHUNK, CH), jnp.int32),
            pltpu.VMEM((NCHUNK, CH), jnp.int32),
            pltpu.VMEM((N,), jnp.float32),
            pltpu.VMEM((N,), jnp.float32),
            pltpu.VMEM((L,), jnp.float32),
            pltpu.VMEM((CH,), jnp.float32),
            pltpu.VMEM((CH, w), jnp.float32),
            pltpu.VMEM((CH, w), jnp.float32),
            pltpu.VMEM((ZCH, w), jnp.float32),
            pltpu.VMEM_SHARED((N, w), jnp.float32),
            pltpu.SemaphoreType.DMA,
            pltpu.SemaphoreType.DMA,
        ],
        compiler_params=pltpu.CompilerParams(needs_layout_passes=False,
                                             use_tc_tiling_on_sc=False),
        name=f"gat_edge_w{w}",
    )
    return fn(haug, alsrc, aldst, bnd, esrc, edst)


def kernel(x, edge_index, W1, a_src1, a_dst1, b1, W2, a_src2, a_dst2, b2,
           Wlin, blin):
    R = 400
    G = N // R

    a2_1 = jnp.stack([a_src1.reshape(HID), a_dst1.reshape(HID)], axis=1)
    a2_2 = jnp.stack([a_src2.reshape(OUT), a_dst2.reshape(OUT)], axis=1)

    haug, alsrc1, aldst1, bnd1 = pl.pallas_call(
        _proj1_body,
        grid=(G,),
        in_specs=[
            pl.BlockSpec((R, IN), lambda i: (i, 0)),
            pl.BlockSpec((IN, HID), lambda i: (0, 0)),
            pl.BlockSpec((HID, 2), lambda i: (0, 0)),
        ],
        out_specs=[
            pl.BlockSpec((R, 80), lambda i: (i, 0)),
            pl.BlockSpec((R, 1), lambda i: (i, 0)),
            pl.BlockSpec((R, 1), lambda i: (i, 0)),
            pl.BlockSpec((1, 16), lambda i: (0, 0)),
        ],
        out_shape=[
            jax.ShapeDtypeStruct((N, 80), jnp.float32),
            jax.ShapeDtypeStruct((N, 1), jnp.float32),
            jax.ShapeDtypeStruct((N, 1), jnp.float32),
            jax.ShapeDtypeStruct((1, 16), jnp.float32),
        ],
    )(x, W1, a2_1)

    esrc = edge_index[0].reshape(NW, NCHUNK, CH)
    edst = edge_index[1].reshape(NW, NCHUNK, CH)
    agg1 = _edge_phase(80, haug, alsrc1.reshape(N), aldst1.reshape(N),
                       bnd1.reshape(16), esrc, edst)

    h2aug, alsrc2, aldst2, bnd2 = pl.pallas_call(
        _combine1_body,
        grid=(G,),
        in_specs=[
            pl.BlockSpec((NC, R, 80), lambda i: (0, i, 0)),
            pl.BlockSpec((1, HID), lambda i: (0, 0)),
            pl.BlockSpec((HID, OUT), lambda i: (0, 0)),
            pl.BlockSpec((OUT, 2), lambda i: (0, 0)),
        ],
        out_specs=[
            pl.BlockSpec((R, 48), lambda i: (i, 0)),
            pl.BlockSpec((R, 1), lambda i: (i, 0)),
            pl.BlockSpec((R, 1), lambda i: (i, 0)),
            pl.BlockSpec((1, 16), lambda i: (0, 0)),
        ],
        out_shape=[
            jax.ShapeDtypeStruct((N, 48), jnp.float32),
            jax.ShapeDtypeStruct((N, 1), jnp.float32),
            jax.ShapeDtypeStruct((N, 1), jnp.float32),
            jax.ShapeDtypeStruct((1, 16), jnp.float32),
        ],
    )(agg1, b1.reshape(1, HID), W2, a2_2)

    agg2 = _edge_phase(48, h2aug, alsrc2.reshape(N), aldst2.reshape(N),
                       bnd2.reshape(16), esrc, edst)

    z, x_ = pl.pallas_call(
        _head_body,
        grid=(G,),
        in_specs=[
            pl.BlockSpec((NC, R, 48), lambda i: (0, i, 0)),
            pl.BlockSpec((1, OUT), lambda i: (0, 0)),
            pl.BlockSpec((OUT, IN), lambda i: (0, 0)),
            pl.BlockSpec((1, IN), lambda i: (0, 0)),
        ],
        out_specs=[
            pl.BlockSpec((R, OUT), lambda i: (i, 0)),
            pl.BlockSpec((R, IN), lambda i: (i, 0)),
        ],
        out_shape=[
            jax.ShapeDtypeStruct((N, OUT), jnp.float32),
            jax.ShapeDtypeStruct((N, IN), jnp.float32),
        ],
    )(agg2, b2.reshape(1, OUT), Wlin, blin.reshape(1, IN))

    B = 2048
    GA = pl.cdiv(N, B)
    zp = jnp.pad(z, ((0, GA * B - N), (0, 0)))
    a_pred = pl.pallas_call(
        _adj_body,
        grid=(GA, GA),
        in_specs=[
            pl.BlockSpec((B, OUT), lambda i, j: (i, 0)),
            pl.BlockSpec((B, OUT), lambda i, j: (j, 0)),
        ],
        out_specs=pl.BlockSpec((B, B), lambda i, j: (i, j)),
        out_shape=jax.ShapeDtypeStruct((N, N), jnp.float32),
    )(zp, zp)

    return (a_pred, z, x_)
